# baseline (speedup 1.0000x reference)
import functools

import jax
import jax.numpy as jnp
from jax import lax
from jax.experimental import pallas as pl
from jax.experimental.pallas import tpu as pltpu
from jax.experimental.pallas import tpu_sc as plsc

N = 10000
E = 320000
D_IN = 128
HID = 96

NC = 2
NS = 16
NW = NC * NS
EW = E // NW
CHUNK = 80
NCHUNK = EW // CHUNK
NTRIPLE = NCHUNK // 3
ACC_N = 10240
ROWS_PER_TILE = ACC_N // NS

HROW = HID
AUXW = 16


def _dense_body(x_ref, w_ref, as_ref, ad_ref, h_ref, asd_ref, adl_ref):
    h = jnp.dot(x_ref[...], w_ref[...],
                preferred_element_type=jnp.float32,
                precision=lax.Precision.HIGHEST)
    h_ref[...] = h
    asd_ref[...] = jnp.dot(h, as_ref[...],
                           preferred_element_type=jnp.float32,
                           precision=lax.Precision.HIGHEST)
    adl_ref[...] = jnp.dot(h, ad_ref[...],
                           preferred_element_type=jnp.float32,
                           precision=lax.Precision.HIGHEST)


def _dense(x, W_gat, att_src, att_dst):
    bn = 2000
    return pl.pallas_call(
        _dense_body,
        grid=(N // bn,),
        in_specs=[
            pl.BlockSpec((bn, D_IN), lambda i: (i, 0)),
            pl.BlockSpec((D_IN, HID), lambda i: (0, 0)),
            pl.BlockSpec((HID, 1), lambda i: (0, 0)),
            pl.BlockSpec((HID, 1), lambda i: (0, 0)),
        ],
        out_specs=[
            pl.BlockSpec((bn, HID), lambda i: (i, 0)),
            pl.BlockSpec((bn, 1), lambda i: (i, 0)),
            pl.BlockSpec((bn, 1), lambda i: (i, 0)),
        ],
        out_shape=[
            jax.ShapeDtypeStruct((N, HID), jnp.float32),
            jax.ShapeDtypeStruct((N, 1), jnp.float32),
            jax.ShapeDtypeStruct((N, 1), jnp.float32),
        ],
    )(x, W_gat, att_src.reshape(HID, 1), att_dst.reshape(HID, 1))


def _edge_body(src_hbm, dst_hbm, as_hbm, ad_hbm, h_hbm,
               s_out, aux_out,
               as_v, ad_v, src_all,
               dst_b0, dst_b1, dst_b2, sdst_v0, sdst_v1, sdst_v2,
               rows_v0, rows_v1, rows_v2, aux_v0, aux_v1, aux_v2,
               w_v0, w_v1, w_v2,
               s_acc, aux_acc, semg0, semg1, semg2, semd0, semd1, semd2,
               sems0r, sems0a, sems1r, sems1a, sems2r, sems2a):
    c = lax.axis_index("c")
    s = lax.axis_index("s")
    wid = s * NC + c

    lane = lax.iota(jnp.int32, 16)
    zero16 = jnp.zeros((16,), jnp.float32)
    unit0 = (lane == 0).astype(jnp.float32)
    unit1 = (lane == 1).astype(jnp.float32)
    col0 = jnp.zeros((16,), jnp.int32)

    def zrow(i, _):
        for j in range(HROW // 16):
            rows_v0[i, pl.ds(j * 16, 16)] = zero16
        aux_v0[i, :] = zero16
        return 0
    lax.fori_loop(0, CHUNK, zrow, 0)
    base_row = s * ROWS_PER_TILE
    for t in range(ROWS_PER_TILE // CHUNK):
        pltpu.sync_copy(rows_v0, s_acc.at[pl.ds(base_row + t * CHUNK, CHUNK)])
        pltpu.sync_copy(aux_v0, aux_acc.at[pl.ds(base_row + t * CHUNK, CHUNK)])

    ebase = wid * EW
    pltpu.sync_copy(as_hbm, as_v)
    pltpu.sync_copy(ad_hbm, ad_v)
    pltpu.sync_copy(src_hbm.at[pl.ds(ebase, EW)], src_all)
    plsc.subcore_barrier()

    bufs0 = (dst_b0, sdst_v0, rows_v0, aux_v0, w_v0, semg0, semd0,
             sems0r, sems0a)
    bufs1 = (dst_b1, sdst_v1, rows_v1, aux_v1, w_v1, semg1, semd1,
             sems1r, sems1a)
    bufs2 = (dst_b2, sdst_v2, rows_v2, aux_v2, w_v2, semg2, semd2,
             sems2r, sems2a)

    def issue_fetch(ci, bufs):
        dst_b, _, rows_v, _, _, semg, semd, _, _ = bufs
        pltpu.async_copy(h_hbm.at[src_all.at[pl.ds(ci * CHUNK, CHUNK)]],
                         rows_v, semg)
        pltpu.async_copy(dst_hbm.at[pl.ds(ebase + ci * CHUNK, CHUNK)],
                         dst_b, semd)

    def wait_fetch(ci, bufs):
        dst_b, _, rows_v, _, _, semg, semd, _, _ = bufs
        pltpu.make_async_copy(h_hbm.at[src_all.at[pl.ds(ci * CHUNK, CHUNK)]],
                              rows_v, semg).wait()
        pltpu.make_async_copy(dst_hbm.at[pl.ds(ebase + ci * CHUNK, CHUNK)],
                              dst_b, semd).wait()

    def compute_chunk(ci, bufs):
        dst_b, sdst_v, rows_v, aux_v, w_v, _, _, _, _ = bufs
        ebase_c = ci * CHUNK
        for g in range(CHUNK // 16):
            s16 = src_all[pl.ds(ebase_c + g * 16, 16)]
            d16 = dst_b[pl.ds(g * 16, 16)]
            av = plsc.load_gather(as_v, [s16])
            dv = plsc.load_gather(ad_v, [d16])
            e = av + dv
            e = jnp.where(e >= 0.0, e, e * 0.2)
            w_v[pl.ds(g * 16, 16)] = jnp.exp(e)
            sdst_v[pl.ds(g * 16, 16)] = d16

        def edge_body(k, _):
            k0 = 2 * k
            k1 = k0 + 1
            wk0 = plsc.load_gather(w_v, [col0 + k0])
            wk1 = plsc.load_gather(w_v, [col0 + k1])
            for j in range(HROW // 16):
                rows_v[k0, pl.ds(j * 16, 16)] = rows_v[k0, pl.ds(j * 16, 16)] * wk0
                rows_v[k1, pl.ds(j * 16, 16)] = rows_v[k1, pl.ds(j * 16, 16)] * wk1
            aux_v[k0, :] = wk0 * unit0 + unit1
            aux_v[k1, :] = wk1 * unit0 + unit1
            return 0
        lax.fori_loop(0, CHUNK // 2, edge_body, 0)

    def issue_scatter(bufs):
        _, sdst_v, rows_v, aux_v, _, _, _, semr, sema = bufs
        d1 = pltpu.async_copy(rows_v, s_acc.at[sdst_v], semr, add=True)
        d2 = pltpu.async_copy(aux_v, aux_acc.at[sdst_v], sema, add=True)
        return d1, d2

    issue_fetch(0, bufs0)
    issue_fetch(1, bufs1)
    issue_fetch(2, bufs2)

    def triple_body(u, _):
        ca = 3 * u
        cb = ca + 1
        cc = ca + 2
        wait_fetch(ca, bufs0)
        compute_chunk(ca, bufs0)
        sa = issue_scatter(bufs0)
        wait_fetch(cb, bufs1)
        compute_chunk(cb, bufs1)
        sb = issue_scatter(bufs1)
        sa[0].wait()
        sa[1].wait()

        @pl.when(ca + 3 < NCHUNK)
        def _():
            issue_fetch(ca + 3, bufs0)
        wait_fetch(cc, bufs2)
        compute_chunk(cc, bufs2)
        sc_ = issue_scatter(bufs2)
        sb[0].wait()
        sb[1].wait()

        @pl.when(cb + 3 < NCHUNK)
        def _():
            issue_fetch(cb + 3, bufs1)
        sc_[0].wait()
        sc_[1].wait()

        @pl.when(cc + 3 < NCHUNK)
        def _():
            issue_fetch(cc + 3, bufs2)
        return 0

    lax.fori_loop(0, NTRIPLE, triple_body, 0)

    for ci, bufs in ((NTRIPLE * 3, bufs0), (NTRIPLE * 3 + 1, bufs1)):
        wait_fetch(ci, bufs)
        compute_chunk(ci, bufs)
        st = issue_scatter(bufs)
        st[0].wait()
        st[1].wait()

    plsc.subcore_barrier()
    pltpu.sync_copy(s_acc.at[pl.ds(base_row, ROWS_PER_TILE)],
                    s_out.at[c, pl.ds(base_row, ROWS_PER_TILE)])
    pltpu.sync_copy(aux_acc.at[pl.ds(base_row, ROWS_PER_TILE)],
                    aux_out.at[c, pl.ds(base_row, ROWS_PER_TILE)])


_edge_kernel = functools.partial(
    pl.kernel,
    out_type=[
        jax.ShapeDtypeStruct((NC, ACC_N, HROW), jnp.float32),
        jax.ShapeDtypeStruct((NC, ACC_N, AUXW), jnp.float32),
    ],
    mesh=plsc.VectorSubcoreMesh(core_axis_name="c", subcore_axis_name="s"),
    compiler_params=pltpu.CompilerParams(use_tc_tiling_on_sc=False,
                                         needs_layout_passes=False),
    scratch_types=[
        pltpu.VMEM((N,), jnp.float32),
        pltpu.VMEM((N,), jnp.float32),
        pltpu.VMEM((EW,), jnp.int32),
        pltpu.VMEM((CHUNK,), jnp.int32),
        pltpu.VMEM((CHUNK,), jnp.int32),
        pltpu.VMEM((CHUNK,), jnp.int32),
        pltpu.VMEM((CHUNK,), jnp.int32),
        pltpu.VMEM((CHUNK,), jnp.int32),
        pltpu.VMEM((CHUNK,), jnp.int32),
        pltpu.VMEM((CHUNK, HROW), jnp.float32),
        pltpu.VMEM((CHUNK, HROW), jnp.float32),
        pltpu.VMEM((CHUNK, HROW), jnp.float32),
        pltpu.VMEM((CHUNK, AUXW), jnp.float32),
        pltpu.VMEM((CHUNK, AUXW), jnp.float32),
        pltpu.VMEM((CHUNK, AUXW), jnp.float32),
        pltpu.VMEM((CHUNK,), jnp.float32),
        pltpu.VMEM((CHUNK,), jnp.float32),
        pltpu.VMEM((CHUNK,), jnp.float32),
        pltpu.VMEM_SHARED((ACC_N, HROW), jnp.float32),
        pltpu.VMEM_SHARED((ACC_N, AUXW), jnp.float32),
        pltpu.SemaphoreType.DMA,
        pltpu.SemaphoreType.DMA,
        pltpu.SemaphoreType.DMA,
        pltpu.SemaphoreType.DMA,
        pltpu.SemaphoreType.DMA,
        pltpu.SemaphoreType.DMA,
        pltpu.SemaphoreType.DMA,
        pltpu.SemaphoreType.DMA,
        pltpu.SemaphoreType.DMA,
        pltpu.SemaphoreType.DMA,
        pltpu.SemaphoreType.DMA,
        pltpu.SemaphoreType.DMA,
    ],
)(_edge_body)


def _norm_body(s_ref, aux_ref, b_ref, o_ref):
    sm = s_ref[0] + s_ref[1]
    aux = aux_ref[0] + aux_ref[1]
    denom = aux[:, 0:1]
    cnt = aux[:, 1:2]
    o_ref[...] = sm / (denom + 1e-16) / jnp.maximum(cnt, 1.0) + b_ref[...]


def _normalize(s_parts, aux_parts, b_gat):
    bn = 2000
    return pl.pallas_call(
        _norm_body,
        grid=(N // bn,),
        in_specs=[
            pl.BlockSpec((NC, bn, HROW), lambda i: (0, i, 0)),
            pl.BlockSpec((NC, bn, AUXW), lambda i: (0, i, 0)),
            pl.BlockSpec((1, HID), lambda i: (0, 0)),
        ],
        out_specs=pl.BlockSpec((bn, HID), lambda i: (i, 0)),
        out_shape=jax.ShapeDtypeStruct((N, HID), jnp.float32),
    )(s_parts, aux_parts, b_gat.reshape(1, HID))


@jax.jit
def kernel(x, edge_index, W_gat, att_src, att_dst, b_gat, W_skip, b_skip):
    h, a_s, a_d = _dense(x, W_gat, att_src, att_dst)
    src = edge_index[0]
    dst = edge_index[1]
    s_parts, aux_parts = _edge_kernel(src, dst, a_s.reshape(N), a_d.reshape(N), h)
    out = _normalize(s_parts, aux_parts, b_gat)
    return out.reshape(3, N, HID // 3).sum(0)


# speedup vs baseline: 1.0630x; 1.0630x over previous
import functools

import jax
import jax.numpy as jnp
from jax import lax
from jax.experimental import pallas as pl
from jax.experimental.pallas import tpu as pltpu
from jax.experimental.pallas import tpu_sc as plsc

N = 10000
E = 320000
D_IN = 128
HID = 96

NC = 2
NS = 16
NW = NC * NS
EW = E // NW
CHUNK = 80
NCHUNK = EW // CHUNK
NTRIPLE = NCHUNK // 3
ACC_N = 10240
ROWS_PER_TILE = ACC_N // NS

HROW = HID
AUXW = 16


def _dense_body(x_ref, w_ref, as_ref, ad_ref, h_ref, asd_ref):
    h = jnp.dot(x_ref[...], w_ref[...],
                preferred_element_type=jnp.float32,
                precision=lax.Precision.HIGHEST)
    h_ref[...] = h
    att2 = jnp.concatenate([as_ref[...], ad_ref[...]], axis=1)
    asd_ref[...] = jnp.dot(h, att2,
                           preferred_element_type=jnp.float32,
                           precision=lax.Precision.HIGHEST)


def _dense(x, W_gat, att_src, att_dst):
    bn = 2000
    return pl.pallas_call(
        _dense_body,
        grid=(N // bn,),
        in_specs=[
            pl.BlockSpec((bn, D_IN), lambda i: (i, 0)),
            pl.BlockSpec((D_IN, HID), lambda i: (0, 0)),
            pl.BlockSpec((HID, 1), lambda i: (0, 0)),
            pl.BlockSpec((HID, 1), lambda i: (0, 0)),
        ],
        out_specs=[
            pl.BlockSpec((bn, HID), lambda i: (i, 0)),
            pl.BlockSpec((bn, 2), lambda i: (i, 0)),
        ],
        out_shape=[
            jax.ShapeDtypeStruct((N, HID), jnp.float32),
            jax.ShapeDtypeStruct((N, 2), jnp.float32),
        ],
    )(x, W_gat, att_src.reshape(HID, 1), att_dst.reshape(HID, 1))


def _edge_body(src_hbm, dst_hbm, as_hbm, ad_hbm, h_hbm,
               s_out, aux_out,
               as_v, ad_v, src_all,
               dst_b0, dst_b1, dst_b2, sdst_v0, sdst_v1, sdst_v2,
               rows_v0, rows_v1, rows_v2, aux_v0, aux_v1, aux_v2,
               w_v0, w_v1, w_v2,
               s_acc, aux_acc, semg0, semg1, semg2, semd0, semd1, semd2,
               sems0r, sems0a, sems1r, sems1a, sems2r, sems2a):
    c = lax.axis_index("c")
    s = lax.axis_index("s")
    wid = s * NC + c

    lane = lax.iota(jnp.int32, 16)
    zero16 = jnp.zeros((16,), jnp.float32)
    unit0 = (lane == 0).astype(jnp.float32)
    unit1 = (lane == 1).astype(jnp.float32)
    col0 = jnp.zeros((16,), jnp.int32)

    def zrow(i, _):
        for j in range(HROW // 16):
            rows_v0[i, pl.ds(j * 16, 16)] = zero16
        aux_v0[i, :] = zero16
        return 0
    lax.fori_loop(0, CHUNK, zrow, 0)
    base_row = s * ROWS_PER_TILE
    for t in range(ROWS_PER_TILE // CHUNK):
        pltpu.sync_copy(rows_v0, s_acc.at[pl.ds(base_row + t * CHUNK, CHUNK)])
        pltpu.sync_copy(aux_v0, aux_acc.at[pl.ds(base_row + t * CHUNK, CHUNK)])

    ebase = wid * EW
    pltpu.sync_copy(as_hbm, as_v)
    pltpu.sync_copy(ad_hbm, ad_v)
    pltpu.sync_copy(src_hbm.at[pl.ds(ebase, EW)], src_all)
    plsc.subcore_barrier()

    bufs0 = (dst_b0, sdst_v0, rows_v0, aux_v0, w_v0, semg0, semd0,
             sems0r, sems0a)
    bufs1 = (dst_b1, sdst_v1, rows_v1, aux_v1, w_v1, semg1, semd1,
             sems1r, sems1a)
    bufs2 = (dst_b2, sdst_v2, rows_v2, aux_v2, w_v2, semg2, semd2,
             sems2r, sems2a)

    def issue_fetch(ci, bufs):
        dst_b, _, rows_v, _, _, semg, semd, _, _ = bufs
        pltpu.async_copy(h_hbm.at[src_all.at[pl.ds(ci * CHUNK, CHUNK)]],
                         rows_v, semg)
        pltpu.async_copy(dst_hbm.at[pl.ds(ebase + ci * CHUNK, CHUNK)],
                         dst_b, semd)

    def wait_fetch(ci, bufs):
        dst_b, _, rows_v, _, _, semg, semd, _, _ = bufs
        pltpu.make_async_copy(h_hbm.at[src_all.at[pl.ds(ci * CHUNK, CHUNK)]],
                              rows_v, semg).wait()
        pltpu.make_async_copy(dst_hbm.at[pl.ds(ebase + ci * CHUNK, CHUNK)],
                              dst_b, semd).wait()

    def compute_chunk(ci, bufs):
        dst_b, sdst_v, rows_v, aux_v, w_v, _, _, _, _ = bufs
        ebase_c = ci * CHUNK
        for g in range(CHUNK // 16):
            s16 = src_all[pl.ds(ebase_c + g * 16, 16)]
            d16 = dst_b[pl.ds(g * 16, 16)]
            av = plsc.load_gather(as_v, [s16])
            dv = plsc.load_gather(ad_v, [d16])
            e = av + dv
            e = jnp.where(e >= 0.0, e, e * 0.2)
            w_v[pl.ds(g * 16, 16)] = jnp.exp(e)
            sdst_v[pl.ds(g * 16, 16)] = d16

        def edge_body(k, _):
            ks = [4 * k, 4 * k + 1, 4 * k + 2, 4 * k + 3]
            wks = [plsc.load_gather(w_v, [col0 + kk]) for kk in ks]
            for j in range(HROW // 16):
                for kk, wkk in zip(ks, wks):
                    rows_v[kk, pl.ds(j * 16, 16)] = (
                        rows_v[kk, pl.ds(j * 16, 16)] * wkk)
            for kk, wkk in zip(ks, wks):
                aux_v[kk, :] = wkk * unit0 + unit1
            return 0
        lax.fori_loop(0, CHUNK // 4, edge_body, 0)

    def issue_scatter(bufs):
        _, sdst_v, rows_v, aux_v, _, _, _, semr, sema = bufs
        d1 = pltpu.async_copy(rows_v, s_acc.at[sdst_v], semr, add=True)
        d2 = pltpu.async_copy(aux_v, aux_acc.at[sdst_v], sema, add=True)
        return d1, d2

    issue_fetch(0, bufs0)
    issue_fetch(1, bufs1)
    issue_fetch(2, bufs2)

    def triple_body(u, _):
        ca = 3 * u
        cb = ca + 1
        cc = ca + 2
        wait_fetch(ca, bufs0)
        compute_chunk(ca, bufs0)
        sa = issue_scatter(bufs0)
        wait_fetch(cb, bufs1)
        compute_chunk(cb, bufs1)
        sb = issue_scatter(bufs1)
        sa[0].wait()
        sa[1].wait()

        @pl.when(ca + 3 < NCHUNK)
        def _():
            issue_fetch(ca + 3, bufs0)
        wait_fetch(cc, bufs2)
        compute_chunk(cc, bufs2)
        sc_ = issue_scatter(bufs2)
        sb[0].wait()
        sb[1].wait()

        @pl.when(cb + 3 < NCHUNK)
        def _():
            issue_fetch(cb + 3, bufs1)
        sc_[0].wait()
        sc_[1].wait()

        @pl.when(cc + 3 < NCHUNK)
        def _():
            issue_fetch(cc + 3, bufs2)
        return 0

    lax.fori_loop(0, NTRIPLE, triple_body, 0)

    for ci, bufs in ((NTRIPLE * 3, bufs0), (NTRIPLE * 3 + 1, bufs1)):
        wait_fetch(ci, bufs)
        compute_chunk(ci, bufs)
        st = issue_scatter(bufs)
        st[0].wait()
        st[1].wait()

    plsc.subcore_barrier()
    pltpu.sync_copy(s_acc.at[pl.ds(base_row, ROWS_PER_TILE)],
                    s_out.at[c, pl.ds(base_row, ROWS_PER_TILE)])
    pltpu.sync_copy(aux_acc.at[pl.ds(base_row, ROWS_PER_TILE)],
                    aux_out.at[c, pl.ds(base_row, ROWS_PER_TILE)])


_edge_kernel = functools.partial(
    pl.kernel,
    out_type=[
        jax.ShapeDtypeStruct((NC, ACC_N, HROW), jnp.float32),
        jax.ShapeDtypeStruct((NC, ACC_N, AUXW), jnp.float32),
    ],
    mesh=plsc.VectorSubcoreMesh(core_axis_name="c", subcore_axis_name="s"),
    compiler_params=pltpu.CompilerParams(use_tc_tiling_on_sc=False,
                                         needs_layout_passes=False),
    scratch_types=[
        pltpu.VMEM((N,), jnp.float32),
        pltpu.VMEM((N,), jnp.float32),
        pltpu.VMEM((EW,), jnp.int32),
        pltpu.VMEM((CHUNK,), jnp.int32),
        pltpu.VMEM((CHUNK,), jnp.int32),
        pltpu.VMEM((CHUNK,), jnp.int32),
        pltpu.VMEM((CHUNK,), jnp.int32),
        pltpu.VMEM((CHUNK,), jnp.int32),
        pltpu.VMEM((CHUNK,), jnp.int32),
        pltpu.VMEM((CHUNK, HROW), jnp.float32),
        pltpu.VMEM((CHUNK, HROW), jnp.float32),
        pltpu.VMEM((CHUNK, HROW), jnp.float32),
        pltpu.VMEM((CHUNK, AUXW), jnp.float32),
        pltpu.VMEM((CHUNK, AUXW), jnp.float32),
        pltpu.VMEM((CHUNK, AUXW), jnp.float32),
        pltpu.VMEM((CHUNK,), jnp.float32),
        pltpu.VMEM((CHUNK,), jnp.float32),
        pltpu.VMEM((CHUNK,), jnp.float32),
        pltpu.VMEM_SHARED((ACC_N, HROW), jnp.float32),
        pltpu.VMEM_SHARED((ACC_N, AUXW), jnp.float32),
        pltpu.SemaphoreType.DMA,
        pltpu.SemaphoreType.DMA,
        pltpu.SemaphoreType.DMA,
        pltpu.SemaphoreType.DMA,
        pltpu.SemaphoreType.DMA,
        pltpu.SemaphoreType.DMA,
        pltpu.SemaphoreType.DMA,
        pltpu.SemaphoreType.DMA,
        pltpu.SemaphoreType.DMA,
        pltpu.SemaphoreType.DMA,
        pltpu.SemaphoreType.DMA,
        pltpu.SemaphoreType.DMA,
    ],
)(_edge_body)


def _norm_body(s_ref, aux_ref, b_ref, o_ref):
    sm = s_ref[0] + s_ref[1]
    aux = aux_ref[0] + aux_ref[1]
    denom = aux[:, 0:1]
    cnt = aux[:, 1:2]
    o_ref[...] = sm / (denom + 1e-16) / jnp.maximum(cnt, 1.0) + b_ref[...]


def _normalize(s_parts, aux_parts, b_gat):
    bn = 2000
    return pl.pallas_call(
        _norm_body,
        grid=(N // bn,),
        in_specs=[
            pl.BlockSpec((NC, bn, HROW), lambda i: (0, i, 0)),
            pl.BlockSpec((NC, bn, AUXW), lambda i: (0, i, 0)),
            pl.BlockSpec((1, HID), lambda i: (0, 0)),
        ],
        out_specs=pl.BlockSpec((bn, HID), lambda i: (i, 0)),
        out_shape=jax.ShapeDtypeStruct((N, HID), jnp.float32),
    )(s_parts, aux_parts, b_gat.reshape(1, HID))


@jax.jit
def kernel(x, edge_index, W_gat, att_src, att_dst, b_gat, W_skip, b_skip):
    h, asd = _dense(x, W_gat, att_src, att_dst)
    src = edge_index[0]
    dst = edge_index[1]
    s_parts, aux_parts = _edge_kernel(src, dst, asd[:, 0], asd[:, 1], h)
    out = _normalize(s_parts, aux_parts, b_gat)
    return out.reshape(3, N, HID // 3).sum(0)
